# baseline (speedup 1.0000x reference)
import functools

import jax
import jax.numpy as jnp
from jax import lax
from jax.experimental import pallas as pl
from jax.experimental.pallas import tpu as pltpu
from jax.experimental.pallas import tpu_sc as plsc

_NUM_CORES = 2
_NUM_SUBCORES = 16
_NUM_WORKERS = _NUM_CORES * _NUM_SUBCORES
_CHUNK = 128
_NBUF = 4
_PAD = 128
_LANES = 16


@functools.cache
def _build(batch: int, seq: int, vocab: int, dim: int):
    bpw = batch // _NUM_WORKERS
    n_sub = bpw // _CHUNK
    n_chunks = seq * n_sub
    n_idx = bpw * seq
    bpw_sh = bpw.bit_length() - 1
    assert (1 << bpw_sh) == bpw and n_sub == _NBUF and n_chunks % _NBUF == 0
    mesh = plsc.VectorSubcoreMesh(core_axis_name="c", subcore_axis_name="s")

    def body(idx_hbm, table_hbm, out_hbm, idx_v, idx_t, rt0, rt1, ss0, ss1,
             *scratch):
        rows = scratch[:_NBUF]
        gsems = scratch[_NBUF:]
        rowst = (rt0, rt1)
        ssems = (ss0, ss1)
        wid = lax.axis_index("s") * _NUM_CORES + lax.axis_index("c")
        b0 = wid * bpw
        lane = jnp.arange(_LANES, dtype=jnp.int32)
        bbv = tuple(bb + lane for bb in range(0, _CHUNK, _LANES))

        pltpu.sync_copy(idx_hbm.at[pl.ds(b0 * seq, n_idx)], idx_v)

        @pl.loop(0, n_idx, step=_LANES)
        def _repack(t0):
            t = t0 + lane
            s_v = t >> bpw_sh
            r_v = t & (bpw - 1)
            vals = plsc.load_gather(idx_v, [r_v * seq + s_v])
            rem = t0 & (bpw - 1)
            idx_t[t0 >> bpw_sh, rem >> 7, pl.ds(rem & (_CHUNK - 1), _LANES)] \
                = vals

        def offsets(k):
            return idx_t.at[k >> 2, k & (n_sub - 1)]

        def out_slice(k):
            return out_hbm.at[k >> 2, :,
                              pl.ds(b0 + (k & (n_sub - 1)) * _CHUNK, _CHUNK)]

        for b in range(_NBUF):
            pltpu.async_copy(table_hbm.at[offsets(b)], rows[b], gsems[b])

        @pl.loop(0, n_chunks, step=_NBUF)
        def _main(k0):
            for m in range(_NBUF):
                k = k0 + m
                rt = rowst[m % 2]
                ss = ssems[m % 2]
                pltpu.make_async_copy(
                    table_hbm.at[offsets(k)], rows[m], gsems[m]
                ).wait()

                @pl.when(k >= 2)
                def _():
                    pltpu.make_async_copy(rt, out_slice(k - 2), ss).wait()

                @pl.loop(0, dim, unroll=8)
                def _tr(d):
                    dv = jnp.full((_LANES,), d, jnp.int32)
                    for i, bv in enumerate(bbv):
                        rt[d, pl.ds(i * _LANES, _LANES)] = plsc.load_gather(
                            rows[m], [bv, dv])

                pltpu.async_copy(rt, out_slice(k), ss)

                @pl.when(k + _NBUF < n_chunks)
                def _():
                    pltpu.async_copy(
                        table_hbm.at[offsets(k + _NBUF)], rows[m], gsems[m]
                    )

        for k in (n_chunks - 2, n_chunks - 1):
            pltpu.make_async_copy(
                rowst[k % 2], out_slice(k), ssems[k % 2]
            ).wait()

    return pl.kernel(
        body,
        out_type=jax.ShapeDtypeStruct((seq, dim, batch), jnp.float32),
        mesh=mesh,
        scratch_types=(
            [
                pltpu.VMEM((n_idx,), jnp.int32),
                pltpu.VMEM((seq, n_sub, _CHUNK), jnp.int32),
                pltpu.VMEM((dim, _CHUNK), jnp.float32),
                pltpu.VMEM((dim, _CHUNK), jnp.float32),
                pltpu.SemaphoreType.DMA,
                pltpu.SemaphoreType.DMA,
            ]
            + [pltpu.VMEM((_CHUNK, _PAD), jnp.float32) for _ in range(_NBUF)]
            + [pltpu.SemaphoreType.DMA for _ in range(_NBUF)]
        ),
        compiler_params=pltpu.CompilerParams(
            use_tc_tiling_on_sc=False, needs_layout_passes=False
        ),
    )


def kernel(inputs, embedding):
    batch, seq = inputs.shape
    vocab, dim = embedding.shape
    assert batch % (_NUM_WORKERS * _CHUNK) == 0, (batch,)
    idx = inputs.reshape(-1).astype(jnp.int32)
    table = jnp.pad(embedding, ((0, 0), (0, _PAD - dim)))
    out = _build(batch, seq, vocab, dim)(idx, table)
    return jnp.transpose(out, (2, 0, 1))


# speedup vs baseline: 1.5159x; 1.5159x over previous
import functools

import jax
import jax.numpy as jnp
from jax import lax
from jax.experimental import pallas as pl
from jax.experimental.pallas import tpu as pltpu
from jax.experimental.pallas import tpu_sc as plsc

_NUM_CORES = 2
_NUM_SUBCORES = 16
_NUM_WORKERS = _NUM_CORES * _NUM_SUBCORES
_CHUNK = 128
_NBUF = 4
_PAD = 128
_LANES = 16


@functools.cache
def _build(batch: int, seq: int, vocab: int, dim: int):
    bpw = batch // _NUM_WORKERS
    n_sub = bpw // _CHUNK
    n_chunks = seq * n_sub
    n_idx = bpw * seq
    bpw_sh = bpw.bit_length() - 1
    assert (1 << bpw_sh) == bpw and n_sub == _NBUF and n_chunks % _NBUF == 0
    mesh = plsc.VectorSubcoreMesh(core_axis_name="c", subcore_axis_name="s")

    def body(idx_hbm, table_hbm, out_hbm, idx_v, idx_t, rt0, rt1, ss0, ss1,
             *scratch):
        rows = scratch[:_NBUF]
        gsems = scratch[_NBUF:]
        rowst = (rt0, rt1)
        ssems = (ss0, ss1)
        wid = lax.axis_index("s") * _NUM_CORES + lax.axis_index("c")
        b0 = wid * bpw
        lane = jnp.arange(_LANES, dtype=jnp.int32)
        dlv = tuple(d0 + lane for d0 in range(0, dim, _LANES))

        pltpu.sync_copy(idx_hbm.at[pl.ds(b0 * seq, n_idx)], idx_v)

        @pl.loop(0, n_idx, step=_LANES)
        def _repack(t0):
            t = t0 + lane
            s_v = t >> bpw_sh
            r_v = t & (bpw - 1)
            vals = plsc.load_gather(idx_v, [r_v * seq + s_v])
            rem = t0 & (bpw - 1)
            idx_t[t0 >> bpw_sh, rem >> 7, pl.ds(rem & (_CHUNK - 1), _LANES)] \
                = vals

        def offsets(k):
            return idx_t.at[k >> 2, k & (n_sub - 1)]

        def out_slice(k):
            return out_hbm.at[k >> 2, :,
                              pl.ds(b0 + (k & (n_sub - 1)) * _CHUNK, _CHUNK)]

        for b in range(_NBUF):
            pltpu.async_copy(table_hbm.at[offsets(b)], rows[b], gsems[b])

        @pl.loop(0, n_chunks, step=_NBUF)
        def _main(k0):
            for m in range(_NBUF):
                k = k0 + m
                rt = rowst[m % 2]
                ss = ssems[m % 2]
                pltpu.make_async_copy(
                    table_hbm.at[offsets(k)], rows[m], gsems[m]
                ).wait()

                @pl.when(k >= 2)
                def _():
                    pltpu.make_async_copy(
                        rt.at[:, pl.ds(0, _CHUNK)], out_slice(k - 2), ss
                    ).wait()

                @pl.loop(0, _CHUNK, unroll=8)
                def _tr(b):
                    bf = jnp.full((_LANES,), b, jnp.int32)
                    for i, dv in enumerate(dlv):
                        plsc.store_scatter(
                            rt, [dv, bf],
                            rows[m][b, pl.ds(i * _LANES, _LANES)],
                        )

                pltpu.async_copy(rt.at[:, pl.ds(0, _CHUNK)], out_slice(k), ss)

                @pl.when(k + _NBUF < n_chunks)
                def _():
                    pltpu.async_copy(
                        table_hbm.at[offsets(k + _NBUF)], rows[m], gsems[m]
                    )

        for k in (n_chunks - 2, n_chunks - 1):
            pltpu.make_async_copy(
                rowst[k % 2].at[:, pl.ds(0, _CHUNK)], out_slice(k),
                ssems[k % 2]
            ).wait()

    return pl.kernel(
        body,
        out_type=jax.ShapeDtypeStruct((seq, dim, batch), jnp.float32),
        mesh=mesh,
        scratch_types=(
            [
                pltpu.VMEM((n_idx,), jnp.int32),
                pltpu.VMEM((seq, n_sub, _CHUNK), jnp.int32),
                pltpu.VMEM((dim, _CHUNK + 5), jnp.float32),
                pltpu.VMEM((dim, _CHUNK + 5), jnp.float32),
                pltpu.SemaphoreType.DMA,
                pltpu.SemaphoreType.DMA,
            ]
            + [pltpu.VMEM((_CHUNK, _PAD), jnp.float32) for _ in range(_NBUF)]
            + [pltpu.SemaphoreType.DMA for _ in range(_NBUF)]
        ),
        compiler_params=pltpu.CompilerParams(
            use_tc_tiling_on_sc=False, needs_layout_passes=False,
            disable_bounds_checks=True,
        ),
    )


def kernel(inputs, embedding):
    batch, seq = inputs.shape
    vocab, dim = embedding.shape
    assert batch % (_NUM_WORKERS * _CHUNK) == 0, (batch,)
    idx = inputs.reshape(-1).astype(jnp.int32)
    table = jnp.pad(embedding, ((0, 0), (0, _PAD - dim)))
    out = _build(batch, seq, vocab, dim)(idx, table)
    return jnp.transpose(out, (2, 0, 1))
